# baseline (speedup 1.0000x reference)
import jax
import jax.numpy as jnp
from jax.experimental import pallas as pl

_BATCH = 4
_SEQ = 8192
_DIM = 1024
_BS = 512


def _tc_body(in_ref, out_ref):
    out_ref[0] = in_ref[...]


def kernel(input_ids, pos_embedding):
    del input_ids
    return pl.pallas_call(
        _tc_body,
        grid=(_SEQ // _BS, _BATCH),
        in_specs=[pl.BlockSpec((_BS, _DIM), lambda i, b: (i, 0))],
        out_specs=pl.BlockSpec((1, _BS, _DIM), lambda i, b: (b, i, 0)),
        out_shape=jax.ShapeDtypeStruct((_BATCH, _SEQ, _DIM), jnp.float32),
    )(pos_embedding)


# speedup vs baseline: 1.4757x; 1.4757x over previous
import jax
import jax.numpy as jnp
from jax.experimental import pallas as pl

_BATCH = 4
_SEQ = 8192
_DIM = 1024
_BS = 512


def _tc_body(in_ref, out_ref):
    row = in_ref[...]
    out_ref[...] = jnp.broadcast_to(row[None], (_BATCH, _BS, _DIM))


def kernel(input_ids, pos_embedding):
    del input_ids
    return pl.pallas_call(
        _tc_body,
        grid=(_SEQ // _BS,),
        in_specs=[pl.BlockSpec((_BS, _DIM), lambda i: (i, 0))],
        out_specs=pl.BlockSpec((_BATCH, _BS, _DIM), lambda i: (0, i, 0)),
        out_shape=jax.ShapeDtypeStruct((_BATCH, _SEQ, _DIM), jnp.float32),
    )(pos_embedding)


# speedup vs baseline: 1.5141x; 1.0260x over previous
import jax
import jax.numpy as jnp
from jax.experimental import pallas as pl

_BATCH = 4
_SEQ = 8192
_DIM = 1024
_BS = 1024


def _tc_body(in_ref, out_ref):
    row = in_ref[...]
    out_ref[...] = jnp.broadcast_to(row[None], (_BATCH, _BS, _DIM))


def kernel(input_ids, pos_embedding):
    del input_ids
    return pl.pallas_call(
        _tc_body,
        grid=(_SEQ // _BS,),
        in_specs=[pl.BlockSpec((_BS, _DIM), lambda i: (i, 0))],
        out_specs=pl.BlockSpec((_BATCH, _BS, _DIM), lambda i: (0, i, 0)),
        out_shape=jax.ShapeDtypeStruct((_BATCH, _SEQ, _DIM), jnp.float32),
    )(pos_embedding)
